# baseline (speedup 1.0000x reference)
import functools

import jax
import jax.numpy as jnp
from jax import lax
from jax.experimental import pallas as pl
from jax.experimental.pallas import tpu as pltpu
from jax.experimental.pallas import tpu_sc as plsc

N_TOKENS = 16384
HALF_DIM = 64
CHUNK = 128

_info = plsc.get_sparse_core_info()
NC, NS = _info.num_cores, _info.num_subcores
NW = 1 * NS
B_PER_W = N_TOKENS // NW
PASS = 512
N_CHUNKS = PASS // CHUNK


def _make_kernel():
    mesh = plsc.VectorSubcoreMesh(core_axis_name="c", subcore_axis_name="s", num_cores=1)

    @functools.partial(
        pl.kernel,
        mesh=mesh,
        out_type=jax.ShapeDtypeStruct((N_TOKENS, 2 * HALF_DIM), jnp.float32),
        scratch_types=[
            pltpu.VMEM((PASS,), jnp.int32),
            pltpu.VMEM((PASS,), jnp.int32),
            pltpu.VMEM((PASS, HALF_DIM), jnp.float32),
            pltpu.VMEM((PASS, HALF_DIM), jnp.float32),
            pltpu.SemaphoreType.DMA,
        ],
        compiler_params=pltpu.CompilerParams(use_tc_tiling_on_sc=False),
    )
    def k(word_idx_hbm, suff_idx_hbm, w_word_hbm, w_suff_hbm, out_hbm,
          idx_w, idx_s, rows_w, rows_s, sem):
        wid = lax.axis_index("s")
        for h in range(B_PER_W // PASS):
            base = wid * B_PER_W + h * PASS
            pltpu.sync_copy(word_idx_hbm.at[pl.ds(base, PASS)], idx_w)
            pltpu.sync_copy(suff_idx_hbm.at[pl.ds(base, PASS)], idx_s)
            gathers = []
            for j in range(N_CHUNKS):
                gathers.append(pltpu.async_copy(
                    w_word_hbm.at[idx_w.at[pl.ds(j * CHUNK, CHUNK)]],
                    rows_w.at[pl.ds(j * CHUNK, CHUNK)], sem))
                gathers.append(pltpu.async_copy(
                    w_suff_hbm.at[idx_s.at[pl.ds(j * CHUNK, CHUNK)]],
                    rows_s.at[pl.ds(j * CHUNK, CHUNK)], sem))
            for c in gathers:
                c.wait()
            pltpu.sync_copy(rows_w, out_hbm.at[pl.ds(base, PASS), pl.ds(0, HALF_DIM)])
            pltpu.sync_copy(rows_s, out_hbm.at[pl.ds(base, PASS), pl.ds(HALF_DIM, HALF_DIM)])

    return k


_sc_lookup = _make_kernel()


def kernel(word_idx, suff_idx, W_word, W_suff):
    return _sc_lookup(word_idx, suff_idx, W_word, W_suff)


# speedup vs baseline: 1.7904x; 1.7904x over previous
import functools

import jax
import jax.numpy as jnp
from jax import lax
from jax.experimental import pallas as pl
from jax.experimental.pallas import tpu as pltpu
from jax.experimental.pallas import tpu_sc as plsc

VOCAB = 1000000
SUFF_PAD = 1024
N_TOKENS = 16384
HALF_DIM = 64

_info = plsc.get_sparse_core_info()
NC, NS = _info.num_cores, _info.num_subcores
NW = NC * NS
B_PER_W = N_TOKENS // NW
PASS = 256
N_PASS = B_PER_W // PASS


def _make_kernel():
    mesh = plsc.VectorSubcoreMesh(core_axis_name="c", subcore_axis_name="s")

    @functools.partial(
        pl.kernel,
        mesh=mesh,
        out_type=jax.ShapeDtypeStruct((N_TOKENS, 2 * HALF_DIM), jnp.float32),
        scratch_types=[
            pltpu.VMEM((B_PER_W,), jnp.int32),
            pltpu.VMEM((B_PER_W,), jnp.int32),
            pltpu.VMEM((8, 8, SUFF_PAD // 128, 128), jnp.float32),
            pltpu.VMEM((8, 8, 128), jnp.float32),
            pltpu.VMEM((8, 8, 128), jnp.float32),
            pltpu.VMEM((PASS, 2 * HALF_DIM), jnp.float32),
            pltpu.SemaphoreType.DMA,
            pltpu.SemaphoreType.DMA,
            pltpu.SemaphoreType.DMA,
        ],
        compiler_params=pltpu.CompilerParams(needs_layout_passes=False),
    )
    def k(word_idx_hbm, suff_idx_hbm, w3_hbm, s4_hbm, out_hbm,
          idx_w, idx_s, suff_v, blk0, blk1, outv, sem0, sem1, ssem):
        wid = lax.axis_index("s") * NC + lax.axis_index("c")
        base = wid * B_PER_W
        scopy = pltpu.make_async_copy(s4_hbm, suff_v, ssem)
        scopy.start()
        pltpu.sync_copy(word_idx_hbm.at[pl.ds(base, B_PER_W)], idx_w)
        pltpu.sync_copy(suff_idx_hbm.at[pl.ds(base, B_PER_W)], idx_s)
        scopy.wait()

        iota = lax.iota(jnp.int32, 16)
        lane_hi = iota >> 3
        lane_lo = iota & 7

        def scalar_at(ref, t):
            grp = (t >> 4) << 4
            lane = t & 15
            v = ref[pl.ds(grp, 16)]
            sel = jnp.where(iota == jnp.full((16,), lane, jnp.int32), v,
                            jnp.zeros((16,), jnp.int32))
            return jnp.sum(sel)

        def start_word(t, blk, sem):
            b = scalar_at(idx_w, t) >> 7
            pltpu.make_async_copy(
                w3_hbm.at[:, :, pl.ds(pl.multiple_of(b * 128, 128), 128)],
                blk, sem).start()

        def wait_blk(blk, sem):
            pltpu.make_async_copy(
                w3_hbm.at[:, :, pl.ds(0, 128)], blk, sem).wait()

        def extract(t, row, blk):
            wl = scalar_at(idx_w, t) & 127
            si = scalar_at(idx_s, t)
            sj = si >> 7
            sl = si & 127
            wl_v = jnp.full((16,), wl, jnp.int32)
            sj_v = jnp.full((16,), sj, jnp.int32)
            sl_v = jnp.full((16,), sl, jnp.int32)
            for cb in range(4):
                d0 = lane_hi + (2 * cb)
                wv = plsc.load_gather(blk, [d0, lane_lo, wl_v])
                outv[row, pl.ds(cb * 16, 16)] = wv
                sv = plsc.load_gather(suff_v, [d0, lane_lo, sj_v, sl_v])
                outv[row, pl.ds(HALF_DIM + cb * 16, 16)] = sv

        for h in range(N_PASS):
            pbase = h * PASS
            start_word(pbase, blk0, sem0)
            start_word(pbase + 1, blk1, sem1)

            def body(i, _):
                t0 = pbase + 2 * i
                wait_blk(blk0, sem0)
                extract(t0, 2 * i, blk0)

                @pl.when(i < PASS // 2 - 1)
                def _():
                    start_word(t0 + 2, blk0, sem0)

                wait_blk(blk1, sem1)
                extract(t0 + 1, 2 * i + 1, blk1)

                @pl.when(i < PASS // 2 - 1)
                def _():
                    start_word(t0 + 3, blk1, sem1)

                return 0

            lax.fori_loop(0, PASS // 2, body, 0)
            pltpu.sync_copy(outv, out_hbm.at[pl.ds(base + pbase, PASS)])

    return k


_sc_lookup = _make_kernel()


def kernel(word_idx, suff_idx, W_word, W_suff):
    w3 = W_word.T.reshape(8, 8, VOCAB)
    sp = jnp.pad(W_suff, ((0, SUFF_PAD - W_suff.shape[0]), (0, 0)))
    s4 = sp.T.reshape(8, 8, SUFF_PAD // 128, 128)
    return _sc_lookup(word_idx.astype(jnp.int32), suff_idx.astype(jnp.int32),
                      w3, s4)


# speedup vs baseline: 2.4675x; 1.3782x over previous
import functools

import jax
import jax.numpy as jnp
from jax import lax
from jax.experimental import pallas as pl
from jax.experimental.pallas import tpu as pltpu
from jax.experimental.pallas import tpu_sc as plsc

VOCAB = 1000000
SUFF_PAD = 1024
N_TOKENS = 16384
HALF_DIM = 64
NBUF = 4

_info = plsc.get_sparse_core_info()
NC, NS = _info.num_cores, _info.num_subcores
NW = NC * NS
B_PER_W = N_TOKENS // NW
PASS = 128
N_PASS = B_PER_W // PASS


def _make_kernel():
    mesh = plsc.VectorSubcoreMesh(core_axis_name="c", subcore_axis_name="s")

    @functools.partial(
        pl.kernel,
        mesh=mesh,
        out_type=jax.ShapeDtypeStruct((N_TOKENS, 2 * HALF_DIM), jnp.float32),
        scratch_types=[
            pltpu.VMEM((B_PER_W,), jnp.int32),
            pltpu.VMEM((B_PER_W,), jnp.int32),
            pltpu.VMEM((8, 8, SUFF_PAD // 128, 128), jnp.float32),
            [pltpu.VMEM((8, 8, 128), jnp.float32) for _ in range(NBUF)],
            pltpu.VMEM((PASS, 2 * HALF_DIM), jnp.float32),
            [pltpu.SemaphoreType.DMA for _ in range(NBUF)],
            pltpu.SemaphoreType.DMA,
        ],
        compiler_params=pltpu.CompilerParams(needs_layout_passes=False),
    )
    def k(word_idx_hbm, suff_idx_hbm, w3_hbm, s4_hbm, out_hbm,
          idx_w, idx_s, suff_v, blks, outv, sems, ssem):
        wid = lax.axis_index("s") * NC + lax.axis_index("c")
        base = wid * B_PER_W
        scopy = pltpu.make_async_copy(s4_hbm, suff_v, ssem)
        scopy.start()
        pltpu.sync_copy(word_idx_hbm.at[pl.ds(base, B_PER_W)], idx_w)
        pltpu.sync_copy(suff_idx_hbm.at[pl.ds(base, B_PER_W)], idx_s)
        scopy.wait()

        iota = lax.iota(jnp.int32, 16)
        lane_hi = iota >> 3
        lane_lo = iota & 7

        def scalar_at(ref, t):
            grp = (t >> 4) << 4
            lane = t & 15
            v = ref[pl.ds(grp, 16)]
            sel = jnp.where(iota == jnp.full((16,), lane, jnp.int32), v,
                            jnp.zeros((16,), jnp.int32))
            return jnp.sum(sel)

        def start_word(t, u):
            b = scalar_at(idx_w, t) >> 7
            pltpu.make_async_copy(
                w3_hbm.at[:, :, pl.ds(pl.multiple_of(b * 128, 128), 128)],
                blks[u], sems[u]).start()

        def wait_blk(u):
            pltpu.make_async_copy(
                w3_hbm.at[:, :, pl.ds(0, 128)], blks[u], sems[u]).wait()

        def extract(t, row, u):
            wl = scalar_at(idx_w, t) & 127
            si = scalar_at(idx_s, t)
            sj = si >> 7
            sl = si & 127
            wl_v = jnp.full((16,), wl, jnp.int32)
            sj_v = jnp.full((16,), sj, jnp.int32)
            sl_v = jnp.full((16,), sl, jnp.int32)
            for cb in range(4):
                d0 = lane_hi + (2 * cb)
                wv = plsc.load_gather(blks[u], [d0, lane_lo, wl_v])
                outv[row, pl.ds(cb * 16, 16)] = wv
                sv = plsc.load_gather(suff_v, [d0, lane_lo, sj_v, sl_v])
                outv[row, pl.ds(HALF_DIM + cb * 16, 16)] = sv

        for h in range(N_PASS):
            pbase = h * PASS
            for u in range(NBUF):
                start_word(pbase + u, u)

            def body(i, _):
                for u in range(NBUF):
                    t = pbase + NBUF * i + u
                    wait_blk(u)
                    extract(t, NBUF * i + u, u)

                    @pl.when(i < PASS // NBUF - 1)
                    def _():
                        start_word(t + NBUF, u)

                return 0

            lax.fori_loop(0, PASS // NBUF, body, 0)
            pltpu.sync_copy(outv, out_hbm.at[pl.ds(base + pbase, PASS)])

    return k


_sc_lookup = _make_kernel()


def kernel(word_idx, suff_idx, W_word, W_suff):
    w3 = W_word.T.reshape(8, 8, VOCAB)
    sp = jnp.pad(W_suff, ((0, SUFF_PAD - W_suff.shape[0]), (0, 0)))
    s4 = sp.T.reshape(8, 8, SUFF_PAD // 128, 128)
    return _sc_lookup(word_idx.astype(jnp.int32), suff_idx.astype(jnp.int32),
                      w3, s4)


# speedup vs baseline: 2.8361x; 1.1494x over previous
import functools

import jax
import jax.numpy as jnp
from jax import lax
from jax.experimental import pallas as pl
from jax.experimental.pallas import tpu as pltpu
from jax.experimental.pallas import tpu_sc as plsc

VOCAB = 1000000
SUFF_PAD = 1024
N_TOKENS = 16384
HALF_DIM = 64
NBUF = 8

_info = plsc.get_sparse_core_info()
NC, NS = _info.num_cores, _info.num_subcores
NW = NC * NS
B_PER_W = N_TOKENS // NW
PASS = 128
N_PASS = B_PER_W // PASS


def _make_kernel():
    mesh = plsc.VectorSubcoreMesh(core_axis_name="c", subcore_axis_name="s")

    @functools.partial(
        pl.kernel,
        mesh=mesh,
        out_type=jax.ShapeDtypeStruct((N_TOKENS, 2 * HALF_DIM), jnp.float32),
        scratch_types=[
            pltpu.VMEM((B_PER_W,), jnp.int32),
            pltpu.VMEM((B_PER_W,), jnp.int32),
            pltpu.VMEM((4, 8, SUFF_PAD // 128, 128), jnp.int32),
            [pltpu.VMEM((8, 8, 128), jnp.float32) for _ in range(NBUF)],
            pltpu.VMEM((PASS, 2 * HALF_DIM), jnp.float32),
            [pltpu.SemaphoreType.DMA for _ in range(NBUF)],
            pltpu.SemaphoreType.DMA,
        ],
        compiler_params=pltpu.CompilerParams(needs_layout_passes=False),
    )
    def k(word_idx_hbm, suff_idx_hbm, w3_hbm, s4_hbm, out_hbm,
          idx_w, idx_s, suff_v, blks, outv, sems, ssem):
        wid = lax.axis_index("s") * NC + lax.axis_index("c")
        base = wid * B_PER_W
        scopy = pltpu.make_async_copy(s4_hbm, suff_v, ssem)
        scopy.start()
        pltpu.sync_copy(word_idx_hbm.at[pl.ds(base, B_PER_W)], idx_w)
        pltpu.sync_copy(suff_idx_hbm.at[pl.ds(base, B_PER_W)], idx_s)
        scopy.wait()

        iota = lax.iota(jnp.int32, 16)
        lane_hi = iota >> 3
        lane_lo = iota & 7
        pair_lo = iota >> 1
        parity = iota & 1
        lo_mask = jnp.full((16,), 0xFFFF, jnp.int32)

        def scalar_at(ref, t):
            grp = (t >> 4) << 4
            lane = t & 15
            v = ref[pl.ds(grp, 16)]
            sel = jnp.where(iota == jnp.full((16,), lane, jnp.int32), v,
                            jnp.zeros((16,), jnp.int32))
            return jnp.sum(sel)

        def start_word(t, u):
            b = scalar_at(idx_w, t) >> 7
            pltpu.make_async_copy(
                w3_hbm.at[:, :, pl.ds(pl.multiple_of(b * 128, 128), 128)],
                blks[u], sems[u]).start()

        def wait_blk(u):
            pltpu.make_async_copy(
                w3_hbm.at[:, :, pl.ds(0, 128)], blks[u], sems[u]).wait()

        def extract(t, row, u):
            wl = scalar_at(idx_w, t) & 127
            si = scalar_at(idx_s, t)
            sj = si >> 7
            sl = si & 127
            wl_v = jnp.full((16,), wl, jnp.int32)
            sj_v = jnp.full((16,), sj, jnp.int32)
            sl_v = jnp.full((16,), sl, jnp.int32)
            for cb in range(4):
                d0 = lane_hi + (2 * cb)
                wv = plsc.load_gather(blks[u], [d0, lane_lo, wl_v])
                outv[row, pl.ds(cb * 16, 16)] = wv
                pw = plsc.load_gather(
                    suff_v, [jnp.full((16,), cb, jnp.int32), pair_lo, sj_v, sl_v])
                half = jnp.where(parity == 1,
                                 lax.shift_right_logical(pw, jnp.full((16,), 16, jnp.int32)),
                                 pw & lo_mask)
                sv = plsc.bitcast(lax.shift_left(half, jnp.full((16,), 16, jnp.int32)),
                                  jnp.float32)
                outv[row, pl.ds(HALF_DIM + cb * 16, 16)] = sv

        for h in range(N_PASS):
            pbase = h * PASS
            for u in range(NBUF):
                start_word(pbase + u, u)

            def body(i, _):
                for u in range(NBUF):
                    t = pbase + NBUF * i + u
                    wait_blk(u)
                    extract(t, NBUF * i + u, u)

                    @pl.when(i < PASS // NBUF - 1)
                    def _():
                        start_word(t + NBUF, u)

                return 0

            lax.fori_loop(0, PASS // NBUF, body, 0)
            pltpu.sync_copy(outv, out_hbm.at[pl.ds(base + pbase, PASS)])

    return k


_sc_lookup = _make_kernel()


def kernel(word_idx, suff_idx, W_word, W_suff):
    w3 = W_word.T.reshape(8, 8, VOCAB)
    sp = jnp.pad(W_suff, ((0, SUFF_PAD - W_suff.shape[0]), (0, 0)))
    spb = sp.astype(jnp.bfloat16).reshape(SUFF_PAD, HALF_DIM // 2, 2)
    packed = jax.lax.bitcast_convert_type(spb, jnp.int32)
    s4 = packed.T.reshape(4, 8, SUFF_PAD // 128, 128)
    return _sc_lookup(word_idx.astype(jnp.int32), suff_idx.astype(jnp.int32),
                      w3, s4)
